# baseline (speedup 1.0000x reference)
import dataclasses
import functools

import jax
import jax.numpy as jnp
from jax import lax
from jax.experimental import pallas as pl
from jax.experimental.pallas import tpu as pltpu
from jax.experimental.pallas import tpu_sc as plsc

B, N, K, D_IN, D_OUT = 4, 8192, 16, 4, 16
NEG = 0.2
TILE = 2048
M = 3 * TILE
NW = 32


def _sc_gather(pts, idx_t, nb):
    pw = nb * N // NW
    nch = pw // 16
    nwb = N // pw
    mesh = plsc.VectorSubcoreMesh(core_axis_name="c", subcore_axis_name="s")
    cp = pltpu.CompilerParams()
    if "needs_layout_passes" in pltpu.CompilerParams.__dataclass_fields__:
        cp = dataclasses.replace(cp, needs_layout_passes=False)

    @functools.partial(
        pl.kernel,
        mesh=mesh,
        compiler_params=cp,
        out_type=jax.ShapeDtypeStruct((nb, K, 3, N), jnp.float32),
        scratch_types=[
            pltpu.VMEM((N * 3,), jnp.float32),
            pltpu.VMEM((K, pw), jnp.int32),
            pltpu.VMEM((K, 3, pw), jnp.float32),
        ],
    )
    def gather_kernel(pts_hbm, idx_hbm, out_hbm, pts_v, idx_v, out_v):
        w = lax.axis_index("s") * 2 + lax.axis_index("c")
        b = w // nwb
        n0 = (w % nwb) * pw
        pltpu.sync_copy(pts_hbm.at[b], pts_v)
        pltpu.sync_copy(idx_hbm.at[b, :, pl.ds(n0, pw)], idx_v)

        @pl.loop(0, nch)
        def _(c):
            c0 = c * 16
            for kk in range(K):
                rows = idx_v[kk, pl.ds(c0, 16)]
                flat = rows * 3
                for t in range(3):
                    out_v[kk, t, pl.ds(c0, 16)] = plsc.load_gather(
                        pts_v, [flat + t])

        pltpu.sync_copy(out_v, out_hbm.at[b, :, :, pl.ds(n0, pw)])

    return gather_kernel(pts, idx_t)


def _leaky_big(x, wdb):
    d = jnp.dot(wdb, x.astype(jnp.bfloat16),
                preferred_element_type=jnp.float32)
    x3 = x.reshape(K, K, M)
    d3 = d.reshape(K, K, M)
    dot = jnp.sum(x3 * d3, axis=0, keepdims=True)
    dns = jnp.sum(d3 * d3, axis=0, keepdims=True)
    f = jnp.where(dot >= 0, 0.0, (1.0 - NEG) * dot / (dns + 1e-7))
    return (x3 - f * d3).reshape(K * K, M)


def _leaky_small(x, wd):
    d = jnp.dot(wd, x, preferred_element_type=jnp.float32)
    dot = jnp.sum(x * d, axis=0, keepdims=True)
    dns = jnp.sum(d * d, axis=0, keepdims=True)
    f = jnp.where(dot >= 0, 0.0, (1.0 - NEG) * dot / (dns + 1e-7))
    return x - f * d


def _tc_body(ngb_ref, ctr_ref, w1b_ref, wd1b_ref, wattb_ref, wmlp_ref,
             wd2_ref, out_ref):
    ngb = ngb_ref[0].reshape(K, M)
    ctrb = jnp.broadcast_to(ctr_ref[...], (K, 3, TILE)).reshape(K, M)
    rel = ctrb - ngb
    rel2 = rel * rel
    dis = jnp.sqrt(rel2[:, :TILE] + rel2[:, TILE:2 * TILE] + rel2[:, 2 * TILE:])
    disb = jnp.concatenate([dis, dis, dis], axis=1)
    feat = jnp.concatenate([disb, rel, ctrb, ngb], axis=0)

    x = jnp.dot(w1b_ref[...], feat.astype(jnp.bfloat16),
                preferred_element_type=jnp.float32)
    x = _leaky_big(x, wd1b_ref[...])

    att = jnp.dot(wattb_ref[...], x.astype(jnp.bfloat16),
                  preferred_element_type=jnp.float32)
    a3 = att.reshape(K, K, M)
    e = jnp.exp(a3 - jnp.max(a3, axis=1, keepdims=True))
    agg = jnp.sum(x.reshape(K, K, M) * e, axis=1) / jnp.sum(e, axis=1)

    o = jnp.dot(wmlp_ref[...], agg, preferred_element_type=jnp.float32)
    o = _leaky_small(o, wd2_ref[...])
    out_ref[...] = o.reshape(1, D_OUT, 3, TILE)


def _tc_call(ngb, ctr_t, w1b, wd1b, wattb, Wmlp, Wd2, nb):
    return pl.pallas_call(
        _tc_body,
        grid=(nb, N // TILE),
        in_specs=[
            pl.BlockSpec((1, K, 3, TILE), lambda b, i: (b, 0, 0, i)),
            pl.BlockSpec((1, 3, TILE), lambda b, i: (b, 0, i)),
            pl.BlockSpec((K * D_OUT, K * D_IN), lambda b, i: (0, 0)),
            pl.BlockSpec((K * D_OUT, K * D_OUT), lambda b, i: (0, 0)),
            pl.BlockSpec((K * D_OUT, K * D_OUT), lambda b, i: (0, 0)),
            pl.BlockSpec((D_OUT, D_OUT), lambda b, i: (0, 0)),
            pl.BlockSpec((D_OUT, D_OUT), lambda b, i: (0, 0)),
        ],
        out_specs=pl.BlockSpec((1, D_OUT, 3, TILE), lambda b, i: (b, 0, 0, i)),
        out_shape=jax.ShapeDtypeStruct((nb, D_OUT, 3, N), jnp.float32),
        compiler_params=pltpu.CompilerParams(
            dimension_semantics=("parallel", "parallel")),
    )(ngb, ctr_t, w1b, wd1b, wattb, Wmlp, Wd2)


@jax.jit
def kernel(pts_xyz, neigh_idx, W1, Wd1, Watt, Wmlp, Wd2):
    idx_t = neigh_idx.astype(jnp.int32).transpose(0, 2, 1)
    pts_f = pts_xyz.reshape(B, N * 3)
    ctr_t = pts_xyz.transpose(0, 2, 1)

    eye = jnp.eye(K, dtype=jnp.float32)
    w1b = jnp.kron(W1, eye).astype(jnp.bfloat16)
    wd1b = jnp.kron(Wd1, eye).astype(jnp.bfloat16)
    wattb = jnp.kron(Watt, eye).astype(jnp.bfloat16)

    ngb = _sc_gather(pts_f, idx_t, B)
    out = _tc_call(ngb, ctr_t, w1b, wd1b, wattb, Wmlp, Wd2, B)
    return out.transpose(0, 3, 1, 2)


# speedup vs baseline: 1.0008x; 1.0008x over previous
import dataclasses
import functools

import jax
import jax.numpy as jnp
from jax import lax
from jax.experimental import pallas as pl
from jax.experimental.pallas import tpu as pltpu
from jax.experimental.pallas import tpu_sc as plsc

B, N, K, D_IN, D_OUT = 4, 8192, 16, 4, 16
NEG = 0.2
TILE = 2048
M = 3 * TILE
NW = 32


def _sc_gather(pts, idx_t, nb):
    pw = nb * N // NW
    nch = pw // 16
    nwb = N // pw
    mesh = plsc.VectorSubcoreMesh(core_axis_name="c", subcore_axis_name="s")
    cp = pltpu.CompilerParams()
    if "needs_layout_passes" in pltpu.CompilerParams.__dataclass_fields__:
        cp = dataclasses.replace(cp, needs_layout_passes=False)

    @functools.partial(
        pl.kernel,
        mesh=mesh,
        compiler_params=cp,
        out_type=jax.ShapeDtypeStruct((nb, K, 3, N), jnp.float32),
        scratch_types=[
            pltpu.VMEM((N * 3,), jnp.float32),
            pltpu.VMEM((K, pw), jnp.int32),
            pltpu.VMEM((K, 3, pw), jnp.float32),
        ],
    )
    def gather_kernel(pts_hbm, idx_hbm, out_hbm, pts_v, idx_v, out_v):
        w = lax.axis_index("s") * 2 + lax.axis_index("c")
        b = w // nwb
        n0 = (w % nwb) * pw
        pltpu.sync_copy(pts_hbm.at[b], pts_v)
        pltpu.sync_copy(idx_hbm.at[b, :, pl.ds(n0, pw)], idx_v)

        @pl.loop(0, nch)
        def _(c):
            c0 = c * 16
            for kk in range(K):
                rows = idx_v[kk, pl.ds(c0, 16)]
                flat = rows * 3
                for t in range(3):
                    out_v[kk, t, pl.ds(c0, 16)] = plsc.load_gather(
                        pts_v, [flat + t])

        pltpu.sync_copy(out_v, out_hbm.at[b, :, :, pl.ds(n0, pw)])

    return gather_kernel(pts, idx_t)


def _leaky_big(x, wdb):
    d = jnp.dot(wdb, x, preferred_element_type=jnp.float32)
    x3 = x.reshape(K, K, M)
    d3 = d.reshape(K, K, M)
    dot = jnp.sum(x3 * d3, axis=0, keepdims=True)
    dns = jnp.sum(d3 * d3, axis=0, keepdims=True)
    f = jnp.where(dot >= 0, 0.0, (1.0 - NEG) * dot / (dns + 1e-7))
    return (x3 - f * d3).reshape(K * K, M)


def _leaky_small(x, wd):
    d = jnp.dot(wd, x, preferred_element_type=jnp.float32)
    dot = jnp.sum(x * d, axis=0, keepdims=True)
    dns = jnp.sum(d * d, axis=0, keepdims=True)
    f = jnp.where(dot >= 0, 0.0, (1.0 - NEG) * dot / (dns + 1e-7))
    return x - f * d


def _tc_body(ngb_ref, ctr_ref, w1b_ref, wd1b_ref, wattb_ref, wmlp_ref,
             wd2_ref, out_ref):
    ngb = ngb_ref[0].reshape(K, M)
    ctrb = jnp.broadcast_to(ctr_ref[...], (K, 3, TILE)).reshape(K, M)
    rel = ctrb - ngb
    rel2 = rel * rel
    dis = jnp.sqrt(rel2[:, :TILE] + rel2[:, TILE:2 * TILE] + rel2[:, 2 * TILE:])
    disb = jnp.concatenate([dis, dis, dis], axis=1)
    feat = jnp.concatenate([disb, rel, ctrb, ngb], axis=0)

    x = jnp.dot(w1b_ref[...], feat, preferred_element_type=jnp.float32)
    x = _leaky_big(x, wd1b_ref[...])

    att = jnp.dot(wattb_ref[...], x, preferred_element_type=jnp.float32)
    a3 = att.reshape(K, K, M)
    e = jnp.exp(a3 - jnp.max(a3, axis=1, keepdims=True))
    agg = jnp.sum(x.reshape(K, K, M) * e, axis=1) / jnp.sum(e, axis=1)

    o = jnp.dot(wmlp_ref[...], agg, preferred_element_type=jnp.float32)
    o = _leaky_small(o, wd2_ref[...])
    out_ref[...] = o.reshape(1, D_OUT, 3, TILE)


def _tc_call(ngb, ctr_t, w1b, wd1b, wattb, Wmlp, Wd2, nb):
    return pl.pallas_call(
        _tc_body,
        grid=(nb, N // TILE),
        in_specs=[
            pl.BlockSpec((1, K, 3, TILE), lambda b, i: (b, 0, 0, i)),
            pl.BlockSpec((1, 3, TILE), lambda b, i: (b, 0, i)),
            pl.BlockSpec((K * D_OUT, K * D_IN), lambda b, i: (0, 0)),
            pl.BlockSpec((K * D_OUT, K * D_OUT), lambda b, i: (0, 0)),
            pl.BlockSpec((K * D_OUT, K * D_OUT), lambda b, i: (0, 0)),
            pl.BlockSpec((D_OUT, D_OUT), lambda b, i: (0, 0)),
            pl.BlockSpec((D_OUT, D_OUT), lambda b, i: (0, 0)),
        ],
        out_specs=pl.BlockSpec((1, D_OUT, 3, TILE), lambda b, i: (b, 0, 0, i)),
        out_shape=jax.ShapeDtypeStruct((nb, D_OUT, 3, N), jnp.float32),
        compiler_params=pltpu.CompilerParams(
            dimension_semantics=("parallel", "parallel")),
    )(ngb, ctr_t, w1b, wd1b, wattb, Wmlp, Wd2)


@jax.jit
def kernel(pts_xyz, neigh_idx, W1, Wd1, Watt, Wmlp, Wd2):
    idx_t = neigh_idx.astype(jnp.int32).transpose(0, 2, 1)
    pts_f = pts_xyz.reshape(B, N * 3)
    ctr_t = pts_xyz.transpose(0, 2, 1)

    eye = jnp.eye(K, dtype=jnp.float32)
    w1b = jnp.kron(W1, eye)
    wd1b = jnp.kron(Wd1, eye)
    wattb = jnp.kron(Watt, eye)

    ngb = _sc_gather(pts_f, idx_t, B)
    out = _tc_call(ngb, ctr_t, w1b, wd1b, wattb, Wmlp, Wd2, B)
    return out.transpose(0, 3, 1, 2)
